# baseline (speedup 1.0000x reference)
import jax
import jax.numpy as jnp
from jax.experimental import pallas as pl
from jax.experimental.pallas import tpu as pltpu


def _copy_body(k_in, v_in, k_out, v_out, sem_k, sem_v):
    ck = pltpu.make_async_copy(k_in, k_out, sem_k)
    cv = pltpu.make_async_copy(v_in, v_out, sem_v)
    ck.start()
    cv.start()
    ck.wait()
    cv.wait()


def kernel(k, v, k_cache, v_cache):
    k_out, v_out = pl.pallas_call(
        _copy_body,
        in_specs=[
            pl.BlockSpec(memory_space=pl.ANY),
            pl.BlockSpec(memory_space=pl.ANY),
        ],
        out_specs=[
            pl.BlockSpec(memory_space=pl.ANY),
            pl.BlockSpec(memory_space=pl.ANY),
        ],
        out_shape=[
            jax.ShapeDtypeStruct(k.shape, k.dtype),
            jax.ShapeDtypeStruct(v.shape, v.dtype),
        ],
        scratch_shapes=[pltpu.SemaphoreType.DMA, pltpu.SemaphoreType.DMA],
    )(k, v)
    return (k_out, v_out)


# speedup vs baseline: 45.9844x; 45.9844x over previous
import functools

import jax
import jax.numpy as jnp
from jax.experimental import pallas as pl
from jax.experimental.pallas import tpu as pltpu

_NBUF = 4
_CHUNK_ROWS = 512


def _copy_body(k_in, v_in, k_out, v_out, *scratch, n_rows):
    bufs = scratch[:_NBUF]
    gsems = scratch[_NBUF:2 * _NBUF]
    ssems = scratch[2 * _NBUF:3 * _NBUF]

    n_per_tensor = n_rows // _CHUNK_ROWS
    chunks = []
    for src, dst in ((k_in, k_out), (v_in, v_out)):
        for c in range(n_per_tensor):
            chunks.append((src, dst, c * _CHUNK_ROWS))
    n = len(chunks)

    gathers = [None] * _NBUF
    for j in range(min(_NBUF, n)):
        src, _, off = chunks[j]
        sl = pl.ds(off, _CHUNK_ROWS)
        gathers[j] = pltpu.async_copy(src.at[sl], bufs[j], gsems[j])

    for j in range(n):
        b = j % _NBUF
        _, dst, off = chunks[j]
        sl = pl.ds(off, _CHUNK_ROWS)
        gathers[b].wait()
        scat = pltpu.async_copy(bufs[b], dst.at[sl], ssems[b])
        jn = j + _NBUF
        if jn < n:
            src_n, _, off_n = chunks[jn]
            sl_n = pl.ds(off_n, _CHUNK_ROWS)
            scat.wait()
            gathers[b] = pltpu.async_copy(src_n.at[sl_n], bufs[b], gsems[b])
        else:
            scat.wait()


def kernel(k, v, k_cache, v_cache):
    L, H, D = k.shape
    body = functools.partial(_copy_body, n_rows=L)
    scratch = (
        [pltpu.VMEM((_CHUNK_ROWS, H, D), k.dtype) for _ in range(_NBUF)]
        + [pltpu.SemaphoreType.DMA for _ in range(2 * _NBUF)]
    )
    k_out, v_out = pl.pallas_call(
        body,
        in_specs=[
            pl.BlockSpec(memory_space=pl.ANY),
            pl.BlockSpec(memory_space=pl.ANY),
        ],
        out_specs=[
            pl.BlockSpec(memory_space=pl.ANY),
            pl.BlockSpec(memory_space=pl.ANY),
        ],
        out_shape=[
            jax.ShapeDtypeStruct(k.shape, k.dtype),
            jax.ShapeDtypeStruct(v.shape, v.dtype),
        ],
        scratch_shapes=scratch,
    )(k, v)
    return (k_out, v_out)
